# baseline (speedup 1.0000x reference)
import jax
import jax.numpy as jnp
from jax.experimental import pallas as pl
from jax.experimental.pallas import tpu as pltpu

_EPS = 1e-12
_NBLK = 512


def _netvlad_block(x_ref, w_ref, b_ref, mut_ref, o_ref, acc_ref):
    n = pl.program_id(1)
    nb = pl.num_programs(1)

    @pl.when(n == 0)
    def _():
        acc_ref[...] = jnp.zeros_like(acc_ref)

    xb = x_ref[0]
    logits = jax.lax.dot_general(
        xb, w_ref[...], (((0,), (0,)), ((), ())),
        preferred_element_type=jnp.float32) + b_ref[...]
    m = jnp.max(logits, axis=1, keepdims=True)
    e = jnp.exp(logits - m)
    a = e / jnp.sum(e, axis=1, keepdims=True)
    x_aug = jnp.concatenate(
        [xb, jnp.ones((8, xb.shape[1]), jnp.float32)], axis=0)
    acc_ref[...] += jax.lax.dot_general(
        x_aug, a, (((1,), (0,)), ((), ())),
        preferred_element_type=jnp.float32)

    @pl.when(n == nb - 1)
    def _():
        acc = acc_ref[...]
        f_dim = acc.shape[0] - 8
        vlad = acc[:f_dim, :] - acc[f_dim:f_dim + 1, :] * mut_ref[...]
        ssq = jnp.sum(vlad * vlad, axis=0, keepdims=True)
        vn = vlad / jnp.maximum(jnp.sqrt(ssq), _EPS)
        gss = jnp.sum(vn * vn, keepdims=True)
        out = vn / jnp.maximum(jnp.sqrt(gss), _EPS)
        o_ref[...] = out[None]


def kernel(x, W, b, mu):
    B, F, N = x.shape
    C = W.shape[1]
    out = pl.pallas_call(
        _netvlad_block,
        out_shape=jax.ShapeDtypeStruct((B, F, C), jnp.float32),
        grid=(B, N // _NBLK),
        in_specs=[
            pl.BlockSpec((1, F, _NBLK), lambda i, j: (i, 0, j)),
            pl.BlockSpec((F, C), lambda i, j: (0, 0)),
            pl.BlockSpec((1, C), lambda i, j: (0, 0)),
            pl.BlockSpec((F, C), lambda i, j: (0, 0)),
        ],
        out_specs=pl.BlockSpec((1, F, C), lambda i, j: (i, 0, 0)),
        scratch_shapes=[pltpu.VMEM((F + 8, C), jnp.float32)],
        compiler_params=pltpu.CompilerParams(
            dimension_semantics=("parallel", "arbitrary"),
        ),
        name="netvlad_fused",
    )(x, W, b.reshape(1, C), mu.T)
    return out.swapaxes(1, 2).reshape(B, C * F)


# speedup vs baseline: 1.5569x; 1.5569x over previous
import jax
import jax.numpy as jnp
from jax.experimental import pallas as pl
from jax.experimental.pallas import tpu as pltpu

_EPS = 1e-12
_NBLK = 1024
_CHUNK = 256


def _netvlad_block(x_ref, w_ref, b_ref, mut_ref, o_ref,
                   acc_ref, asum_ref, a_ref):
    n = pl.program_id(1)
    nb = pl.num_programs(1)

    @pl.when(n == 0)
    def _():
        acc_ref[...] = jnp.zeros_like(acc_ref)
        asum_ref[...] = jnp.zeros_like(asum_ref)

    xb = x_ref[0]
    w = w_ref[...]
    bias = b_ref[...]
    asum = jnp.zeros_like(asum_ref)
    for k in range(_NBLK // _CHUNK):
        xc = xb[:, k * _CHUNK:(k + 1) * _CHUNK]
        logits = jax.lax.dot_general(
            xc, w, (((0,), (0,)), ((), ())),
            preferred_element_type=jnp.float32) + bias
        m = jnp.max(logits, axis=1, keepdims=True)
        e = jnp.exp(logits - m)
        a = e / jnp.sum(e, axis=1, keepdims=True)
        a_ref[k * _CHUNK:(k + 1) * _CHUNK, :] = a
        asum = asum + jnp.sum(a, axis=0, keepdims=True)
    asum_ref[...] += asum

    acc_ref[...] += jax.lax.dot_general(
        xb, a_ref[...], (((1,), (0,)), ((), ())),
        preferred_element_type=jnp.float32)

    @pl.when(n == nb - 1)
    def _():
        vlad = acc_ref[...] - asum_ref[...] * mut_ref[...]
        ssq = jnp.sum(vlad * vlad, axis=0, keepdims=True)
        vn = vlad / jnp.maximum(jnp.sqrt(ssq), _EPS)
        gss = jnp.sum(vn * vn, keepdims=True)
        out = vn / jnp.maximum(jnp.sqrt(gss), _EPS)
        o_ref[...] = out[None]


def kernel(x, W, b, mu):
    B, F, N = x.shape
    C = W.shape[1]
    out = pl.pallas_call(
        _netvlad_block,
        out_shape=jax.ShapeDtypeStruct((B, F, C), jnp.float32),
        grid=(B, N // _NBLK),
        in_specs=[
            pl.BlockSpec((1, F, _NBLK), lambda i, j: (i, 0, j)),
            pl.BlockSpec((F, C), lambda i, j: (0, 0)),
            pl.BlockSpec((1, C), lambda i, j: (0, 0)),
            pl.BlockSpec((F, C), lambda i, j: (0, 0)),
        ],
        out_specs=pl.BlockSpec((1, F, C), lambda i, j: (i, 0, 0)),
        scratch_shapes=[
            pltpu.VMEM((F, C), jnp.float32),
            pltpu.VMEM((1, C), jnp.float32),
            pltpu.VMEM((_NBLK, C), jnp.float32),
        ],
        compiler_params=pltpu.CompilerParams(
            dimension_semantics=("parallel", "arbitrary"),
        ),
        name="netvlad_fused",
    )(x, W, b.reshape(1, C), mu.T)
    return out.swapaxes(1, 2).reshape(B, C * F)
